# speedup vs baseline: 6.3767x; 6.3767x over previous
import functools

import jax
import jax.numpy as jnp
from jax import lax
from jax.experimental import pallas as pl
from jax.experimental.pallas import tpu as pltpu

GROUP_SIZE = 32
BQ = 256


def _knn_group_kernel(xyzq_ref, xyz_ref, feat_ref, neigh_ref, nfeat_ref,
                      dist_ref, *, n, k):
    q = xyzq_ref[0]
    ka = xyz_ref[0]
    sqq = jnp.sum(q * q, axis=1)
    sqk = jnp.sum(ka * ka, axis=1)
    inner = lax.dot_general(q, ka, (((1,), (1,)), ((), ())),
                            preferred_element_type=jnp.float32)
    dist_ref[...] = (sqq[:, None] + sqk[None, :]) - 2.0 * inner

    iota_n = lax.broadcasted_iota(jnp.int32, (BQ, n), 1)

    def body(j, _):
        dist = dist_ref[...]
        m = jnp.min(dist, axis=1, keepdims=True)
        sel = jnp.min(jnp.where(dist == m, iota_n, n), axis=1)
        onehot = iota_n == sel[:, None]
        oh = onehot.astype(jnp.float32)
        xyz_sel = lax.dot_general(oh, ka, (((1,), (0,)), ((), ())),
                                  preferred_element_type=jnp.float32)
        feat_sel = lax.dot_general(oh, feat_ref[0], (((1,), (0,)), ((), ())),
                                   preferred_element_type=jnp.float32)
        neigh_ref[0, :, j, :] = xyz_sel - q
        nfeat_ref[0, :, j, :] = feat_sel
        dist_ref[...] = jnp.where(onehot, jnp.inf, dist)
        return 0

    lax.fori_loop(0, k, body, 0)


def kernel(xyz, feat):
    b, n, _ = xyz.shape
    c = feat.shape[-1]
    k = GROUP_SIZE
    grid = (b, n // BQ)
    out_shapes = (
        jax.ShapeDtypeStruct((b, n, k, 3), jnp.float32),
        jax.ShapeDtypeStruct((b, n, k, c), jnp.float32),
    )
    neigh, nfeat = pl.pallas_call(
        functools.partial(_knn_group_kernel, n=n, k=k),
        grid=grid,
        in_specs=[
            pl.BlockSpec((1, BQ, 3), lambda bi, qi: (bi, qi, 0)),
            pl.BlockSpec((1, n, 3), lambda bi, qi: (bi, 0, 0)),
            pl.BlockSpec((1, n, c), lambda bi, qi: (bi, 0, 0)),
        ],
        out_specs=(
            pl.BlockSpec((1, BQ, k, 3), lambda bi, qi: (bi, qi, 0, 0)),
            pl.BlockSpec((1, BQ, k, c), lambda bi, qi: (bi, qi, 0, 0)),
        ),
        out_shape=out_shapes,
        scratch_shapes=[pltpu.VMEM((BQ, n), jnp.float32)],
    )(xyz, xyz, feat)
    return (neigh, nfeat)


# speedup vs baseline: 7.8902x; 1.2374x over previous
import functools

import jax
import jax.numpy as jnp
from jax import lax
from jax.experimental import pallas as pl
from jax.experimental.pallas import tpu as pltpu
from jax.experimental.pallas import tpu_sc as plsc

GROUP_SIZE = 32
BQ = 256

NC = 2
NS = 16
NW = NC * NS
SC_CHUNK = 128
SC_NBUF = 2


def _knn_kernel(xyzq_ref, xyz_ref, neigh_ref, gidx_ref, dist_ref, *, n, k):
    q = xyzq_ref[0]
    ka = xyz_ref[0]
    sqq = jnp.sum(q * q, axis=1)
    sqk = jnp.sum(ka * ka, axis=1)
    inner = lax.dot_general(q, ka, (((1,), (1,)), ((), ())),
                            preferred_element_type=jnp.float32)
    dist_ref[...] = (sqq[:, None] + sqk[None, :]) - 2.0 * inner

    iota_n = lax.broadcasted_iota(jnp.int32, (BQ, n), 1)
    boff = pl.program_id(0) * n

    def body(j, _):
        dist = dist_ref[...]
        m = jnp.min(dist, axis=1, keepdims=True)
        sel = jnp.min(jnp.where(dist == m, iota_n, n), axis=1)
        onehot = iota_n == sel[:, None]
        oh = onehot.astype(jnp.float32)
        xyz_sel = lax.dot_general(oh, ka, (((1,), (0,)), ((), ())),
                                  preferred_element_type=jnp.float32)
        neigh_ref[0, :, j, :] = xyz_sel - q
        gidx_ref[0, pl.ds(j, 1), :] = (sel + boff)[None, :]
        dist_ref[...] = jnp.where(onehot, jnp.inf, dist)
        return 0

    lax.fori_loop(0, k, body, 0)


def _knn_tc(xyz):
    b, n, _ = xyz.shape
    k = GROUP_SIZE
    grid = (b, n // BQ)
    out_shapes = (
        jax.ShapeDtypeStruct((b, n, k, 3), jnp.float32),
        jax.ShapeDtypeStruct((b, k, n), jnp.int32),
    )
    return pl.pallas_call(
        functools.partial(_knn_kernel, n=n, k=k),
        grid=grid,
        in_specs=[
            pl.BlockSpec((1, BQ, 3), lambda bi, qi: (bi, qi, 0)),
            pl.BlockSpec((1, n, 3), lambda bi, qi: (bi, 0, 0)),
        ],
        out_specs=(
            pl.BlockSpec((1, BQ, k, 3), lambda bi, qi: (bi, qi, 0, 0)),
            pl.BlockSpec((1, k, BQ), lambda bi, qi: (bi, 0, qi)),
        ),
        out_shape=out_shapes,
        scratch_shapes=[pltpu.VMEM((BQ, n), jnp.float32)],
    )(xyz, xyz)


def _sc_gather(feat_flat, gidx_flat):
    rtot = gidx_flat.shape[0]
    c = feat_flat.shape[-1]
    rw = rtot // NW
    nchunk = rw // SC_CHUNK

    mesh = plsc.VectorSubcoreMesh(core_axis_name="c", subcore_axis_name="s",
                                  num_cores=NC, num_subcores=NS)

    @functools.partial(
        pl.kernel, mesh=mesh,
        out_type=jax.ShapeDtypeStruct((rtot, c), jnp.float32),
        scratch_types=[
            pltpu.VMEM((rw,), jnp.int32),
            pltpu.VMEM((SC_CHUNK, c), jnp.float32),
            pltpu.VMEM((SC_CHUNK, c), jnp.float32),
            pltpu.SemaphoreType.DMA,
            pltpu.SemaphoreType.DMA,
        ],
    )
    def gather(feat_hbm, idx_hbm, out_hbm, idx_v, r0, r1, s0, s1):
        wid = lax.axis_index("s") * NC + lax.axis_index("c")
        base = wid * rw
        pltpu.sync_copy(idx_hbm.at[pl.ds(base, rw)], idx_v)
        bufs = (r0, r1)
        sems = (s0, s1)
        for bslot in range(SC_NBUF):
            pltpu.async_copy(
                feat_hbm.at[idx_v.at[pl.ds(bslot * SC_CHUNK, SC_CHUNK)]],
                bufs[bslot], sems[bslot])

        @pl.loop(0, nchunk, step=SC_NBUF)
        def _(g0):
            for bslot in range(SC_NBUF):
                g = g0 + bslot
                pltpu.make_async_copy(
                    feat_hbm.at[idx_v.at[pl.ds(g * SC_CHUNK, SC_CHUNK)]],
                    bufs[bslot], sems[bslot]).wait()
                pltpu.sync_copy(bufs[bslot],
                                out_hbm.at[pl.ds(base + g * SC_CHUNK, SC_CHUNK)])

                @pl.when(g + SC_NBUF < nchunk)
                def _fire():
                    pltpu.async_copy(
                        feat_hbm.at[
                            idx_v.at[pl.ds((g + SC_NBUF) * SC_CHUNK, SC_CHUNK)]],
                        bufs[bslot], sems[bslot])

    return gather(feat_flat, gidx_flat)


def kernel(xyz, feat):
    b, n, _ = xyz.shape
    c = feat.shape[-1]
    k = GROUP_SIZE
    neigh, gidx = _knn_tc(xyz)
    gidx_flat = gidx.transpose(0, 2, 1).reshape(b * n * k)
    nfeat = _sc_gather(feat.reshape(b * n, c), gidx_flat)
    return (neigh, nfeat.reshape(b, n, k, c))


# speedup vs baseline: 9.7903x; 1.2408x over previous
import functools

import jax
import jax.numpy as jnp
from jax import lax
from jax.experimental import pallas as pl
from jax.experimental.pallas import tpu as pltpu
from jax.experimental.pallas import tpu_sc as plsc

GROUP_SIZE = 32
BQ = 256

NC = 2
NS = 16
NW = NC * NS
SC_CHUNK = 64


def _knn_kernel(xyzq_ref, xyz_ref, gidx_ref, dist_ref, *, n, k):
    q = xyzq_ref[0]
    ka = xyz_ref[0]
    sqq = jnp.sum(q * q, axis=1)
    sqk = jnp.sum(ka * ka, axis=1)
    inner = lax.dot_general(q, ka, (((1,), (1,)), ((), ())),
                            preferred_element_type=jnp.float32)
    dist_ref[...] = (sqq[:, None] + sqk[None, :]) - 2.0 * inner

    iota_n = lax.broadcasted_iota(jnp.int32, (BQ, n), 1)
    boff = pl.program_id(0) * n

    def body(j, _):
        dist = dist_ref[...]
        m = jnp.min(dist, axis=1, keepdims=True)
        sel = jnp.min(jnp.where(dist == m, iota_n, n), axis=1)
        gidx_ref[0, pl.ds(j, 1), :] = (sel + boff)[None, :]
        dist_ref[...] = jnp.where(iota_n == sel[:, None], jnp.inf, dist)
        return 0

    lax.fori_loop(0, k, body, 0)


def _knn_tc(xyz):
    b, n, _ = xyz.shape
    k = GROUP_SIZE
    grid = (b, n // BQ)
    return pl.pallas_call(
        functools.partial(_knn_kernel, n=n, k=k),
        grid=grid,
        in_specs=[
            pl.BlockSpec((1, BQ, 3), lambda bi, qi: (bi, qi, 0)),
            pl.BlockSpec((1, n, 3), lambda bi, qi: (bi, 0, 0)),
        ],
        out_specs=pl.BlockSpec((1, k, BQ), lambda bi, qi: (bi, 0, qi)),
        out_shape=jax.ShapeDtypeStruct((b, k, n), jnp.int32),
        scratch_shapes=[pltpu.VMEM((BQ, n), jnp.float32)],
    )(xyz, xyz)


def _sc_gather(feat_flat, xyz, gidx):
    b, k, n = gidx.shape
    c = feat_flat.shape[-1]
    rtot = b * n * k
    rw = rtot // NW
    pw = rw // k
    wpb = n // pw
    nchunk = rw // SC_CHUNK
    ppc = SC_CHUNK // k
    ngrp = SC_CHUNK // 16

    mesh = plsc.VectorSubcoreMesh(core_axis_name="c", subcore_axis_name="s",
                                  num_cores=NC, num_subcores=NS)

    @functools.partial(
        pl.kernel, mesh=mesh,
        compiler_params=pltpu.CompilerParams(needs_layout_passes=False),
        out_type=(
            jax.ShapeDtypeStruct((rtot, c), jnp.float32),
            jax.ShapeDtypeStruct((rtot * 3,), jnp.float32),
        ),
        scratch_types=[
            pltpu.VMEM((k * pw,), jnp.int32),
            pltpu.VMEM((n * 3,), jnp.float32),
            pltpu.VMEM((rw * 3,), jnp.float32),
            pltpu.VMEM((SC_CHUNK,), jnp.int32),
            pltpu.VMEM((SC_CHUNK,), jnp.int32),
            pltpu.VMEM((SC_CHUNK, c), jnp.float32),
            pltpu.VMEM((SC_CHUNK, c), jnp.float32),
            pltpu.SemaphoreType.DMA,
            pltpu.SemaphoreType.DMA,
        ],
    )
    def body(feat_hbm, xyz_hbm, idx_hbm, feat_out, neigh_out,
             slab_v, xyz_v, nst_v, ir0, ir1, fb0, fb1, s0, s1):
        wid = lax.axis_index("s") * NC + lax.axis_index("c")
        bi = wid // wpb
        col0 = (wid % wpb) * pw
        base = wid * rw
        boff = bi * n

        for kk in range(k):
            pltpu.sync_copy(idx_hbm.at[bi, kk, pl.ds(col0, pw)],
                            slab_v.at[pl.ds(kk * pw, pw)])
        pltpu.sync_copy(xyz_hbm.at[bi], xyz_v)

        iota16 = lax.broadcasted_iota(jnp.int32, (16,), 0)
        zer16 = jnp.zeros((16,), jnp.int32)
        irs = (ir0, ir1)
        fbs = (fb0, fb1)
        sems = (s0, s1)

        def build_chunk(g, ir):
            for t in range(ngrp):
                kv = (t % 2) * 16 + iota16
                p = g * ppc + t // 2
                gv = plsc.load_gather(slab_v, [kv * pw + p])
                ir[pl.ds(16 * t, 16)] = gv
                nloc = gv - boff
                cp = col0 + p
                rl = g * SC_CHUNK + 16 * t + iota16
                for d in range(3):
                    xs = plsc.load_gather(xyz_v, [nloc * 3 + d])
                    cs = plsc.load_gather(xyz_v, [zer16 + (cp * 3 + d)])
                    plsc.store_scatter(nst_v, [rl * 3 + d], xs - cs)

        @pl.loop(0, nchunk, step=2)
        def _(g0):
            for bslot in range(2):
                g = g0 + bslot
                build_chunk(g, irs[bslot])
                pltpu.async_copy(feat_hbm.at[irs[bslot]], fbs[bslot],
                                 sems[bslot])

                @pl.when(g >= 1)
                def _drain():
                    other = 1 - bslot
                    pltpu.make_async_copy(feat_hbm.at[irs[other]], fbs[other],
                                          sems[other]).wait()
                    pltpu.sync_copy(
                        fbs[other],
                        feat_out.at[pl.ds(base + (g - 1) * SC_CHUNK, SC_CHUNK)])

        last = nchunk - 1
        lslot = last % 2
        pltpu.make_async_copy(feat_hbm.at[irs[lslot]], fbs[lslot],
                              sems[lslot]).wait()
        pltpu.sync_copy(fbs[lslot],
                        feat_out.at[pl.ds(base + last * SC_CHUNK, SC_CHUNK)])
        pltpu.sync_copy(nst_v, neigh_out.at[pl.ds(base * 3, rw * 3)])

    return body(feat_flat, xyz.reshape(b, n * 3), gidx)


def kernel(xyz, feat):
    b, n, _ = xyz.shape
    c = feat.shape[-1]
    k = GROUP_SIZE
    gidx = _knn_tc(xyz)
    nfeat, neigh = _sc_gather(feat.reshape(b * n, c), xyz, gidx)
    return (neigh.reshape(b, n, k, 3), nfeat.reshape(b, n, k, c))
